# baseline (speedup 1.0000x reference)
import functools

import jax
import jax.numpy as jnp
from jax import lax
from jax.experimental import pallas as pl
from jax.experimental.pallas import tpu as pltpu
from jax.experimental.pallas import tpu_sc as plsc

EMBED_DIM = 64


def _info():
    info = plsc.get_sparse_core_info()
    return info.num_cores, info.num_subcores, info.num_lanes


@functools.lru_cache(maxsize=None)
def _make_transpose(V: int):
    D = EMBED_DIM
    NC, NS, L = _info()
    NW = NC * NS
    W = 512
    nblk = V // W
    TAIL = V - nblk * W
    assert TAIL % L == 0

    mesh = plsc.VectorSubcoreMesh(core_axis_name="c", subcore_axis_name="s")

    @functools.partial(
        pl.kernel,
        mesh=mesh,
        out_type=jax.ShapeDtypeStruct((V, 2 * D), jnp.float32),
        scratch_types=[
            pltpu.VMEM((D, W), jnp.float32),
            pltpu.VMEM((W, 2 * D), jnp.float32),
            pltpu.VMEM((D, TAIL), jnp.float32),
        ],
        compiler_params=pltpu.CompilerParams(needs_layout_passes=False),
    )
    def transpose_kernel(tT_hbm, tailT_hbm, out_hbm, slab_v, stage_v, tail_v):
        wid = lax.axis_index("s") * NC + lax.axis_index("c")
        iota = lax.iota(jnp.int32, L)

        def transpose_rows(src_v, n):
            def row(c, _):
                cvec = iota * 0 + c
                for k in range(D // L):
                    d16 = iota + L * k
                    v = plsc.load_gather(src_v, [d16, cvec])
                    plsc.store_scatter(stage_v, [cvec, d16], v)
                return _
            lax.fori_loop(0, n, row, None)

        def blk_body(blk, _):
            c0 = blk * W
            pltpu.sync_copy(tT_hbm.at[:, pl.ds(c0, W)], slab_v)
            transpose_rows(slab_v, W)
            pltpu.sync_copy(stage_v, out_hbm.at[pl.ds(c0, W)])
            return _

        n_mine = (nblk - wid + NW - 1) // NW
        lax.fori_loop(0, n_mine, lambda i, _: blk_body(wid + i * NW, _), None)

        if TAIL:
            @pl.when(wid == 0)
            def _tail():
                pltpu.sync_copy(tailT_hbm, tail_v)
                transpose_rows(tail_v, TAIL)
                pltpu.sync_copy(
                    stage_v.at[pl.ds(0, TAIL)], out_hbm.at[pl.ds(nblk * W, TAIL)]
                )

    return transpose_kernel


@functools.lru_cache(maxsize=None)
def _make_gather(nb: int, ns_pad: int):
    D = EMBED_DIM
    NC, NS, L = _info()
    NW = NC * NS
    assert nb % NW == 0
    b_per_w = nb // NW
    GB = 8
    CH = GB * ns_pad
    n_ch = b_per_w // GB
    assert n_ch * GB == b_per_w

    mesh = plsc.VectorSubcoreMesh(core_axis_name="c", subcore_axis_name="s")

    @functools.partial(
        pl.kernel,
        mesh=mesh,
        out_type=jax.ShapeDtypeStruct((nb, ns_pad, 2 * D), jnp.float32),
        scratch_types=[
            pltpu.VMEM((b_per_w * ns_pad,), jnp.int32),
            pltpu.VMEM((CH, 2 * D), jnp.float32),
            pltpu.SemaphoreType.DMA,
        ],
        compiler_params=pltpu.CompilerParams(needs_layout_passes=False),
    )
    def gather_kernel(t128_hbm, idx_hbm, out_hbm, idx_v, rows_v, sem):
        wid = lax.axis_index("s") * NC + lax.axis_index("c")
        base = wid * b_per_w * ns_pad
        pltpu.sync_copy(idx_hbm.at[pl.ds(base, b_per_w * ns_pad)], idx_v)

        def chunk(i, _):
            pltpu.async_copy(
                t128_hbm.at[idx_v.at[pl.ds(i * CH, CH)]], rows_v, sem
            ).wait()
            b0 = wid * b_per_w + i * GB
            for k in range(GB):
                pltpu.sync_copy(
                    rows_v.at[pl.ds(k * ns_pad, ns_pad)], out_hbm.at[b0 + k]
                )
            return _
        lax.fori_loop(0, n_ch, chunk, None)

    return gather_kernel


def kernel(table, inputs):
    nb, ns = inputs.shape
    D = table.shape[1]
    ns_pad = (ns + 7) // 8 * 8
    idxp = jnp.pad(inputs.astype(jnp.int32), ((0, 0), (0, ns_pad - ns)))
    idx = idxp.reshape(-1)
    V = table.shape[0]
    tT = table.T
    ntail = V % 512
    tailT = lax.slice(tT, (0, V - ntail), (D, V))
    t128 = _make_transpose(V)(tT, tailT)
    out3 = _make_gather(nb, ns_pad)(t128, idx)
    return out3[:, :ns, :D]


# speedup vs baseline: 1.1533x; 1.1533x over previous
import functools

import jax
import jax.numpy as jnp
from jax import lax
from jax.experimental import pallas as pl
from jax.experimental.pallas import tpu as pltpu
from jax.experimental.pallas import tpu_sc as plsc

EMBED_DIM = 64


def _info():
    info = plsc.get_sparse_core_info()
    return info.num_cores, info.num_subcores, info.num_lanes


@functools.lru_cache(maxsize=None)
def _make_transpose(V: int):
    D = EMBED_DIM
    NC, NS, L = _info()
    NW = NC * NS
    W = 512
    nblk = V // W
    TAIL = V - nblk * W
    assert TAIL % L == 0

    mesh = plsc.VectorSubcoreMesh(core_axis_name="c", subcore_axis_name="s")

    @functools.partial(
        pl.kernel,
        mesh=mesh,
        out_type=jax.ShapeDtypeStruct((V, 2 * D), jnp.float32),
        scratch_types=[
            pltpu.VMEM((D, W), jnp.float32),
            pltpu.VMEM((W, 2 * D), jnp.float32),
            pltpu.VMEM((D, TAIL), jnp.float32),
        ],
        compiler_params=pltpu.CompilerParams(needs_layout_passes=False),
    )
    def transpose_kernel(tT_hbm, tailT_hbm, out_hbm, slab_v, stage_v, tail_v):
        wid = lax.axis_index("s") * NC + lax.axis_index("c")
        iota = lax.iota(jnp.int32, L)

        def transpose_rows(src_v, n):
            @plsc.parallel_loop(0, n // L, unroll=4)
            def col_group(g):
                rvec = g * L + iota
                for d in range(D):
                    v = src_v[d, pl.ds(g * L, L)]
                    plsc.store_scatter(stage_v, [rvec, iota * 0 + d], v)

        def blk_body(blk, _):
            c0 = blk * W
            pltpu.sync_copy(tT_hbm.at[:, pl.ds(c0, W)], slab_v)
            transpose_rows(slab_v, W)
            pltpu.sync_copy(stage_v, out_hbm.at[pl.ds(c0, W)])
            return _

        n_mine = (nblk - wid + NW - 1) // NW
        lax.fori_loop(0, n_mine, lambda i, _: blk_body(wid + i * NW, _), None)

        if TAIL:
            @pl.when(wid == 0)
            def _tail():
                pltpu.sync_copy(tailT_hbm, tail_v)
                transpose_rows(tail_v, TAIL)
                pltpu.sync_copy(
                    stage_v.at[pl.ds(0, TAIL)], out_hbm.at[pl.ds(nblk * W, TAIL)]
                )

    return transpose_kernel


@functools.lru_cache(maxsize=None)
def _make_gather(nb: int, ns_pad: int):
    D = EMBED_DIM
    NC, NS, L = _info()
    NW = NC * NS
    assert nb % NW == 0
    b_per_w = nb // NW
    GB = 8
    CH = GB * ns_pad
    n_ch = b_per_w // GB
    assert n_ch * GB == b_per_w

    mesh = plsc.VectorSubcoreMesh(core_axis_name="c", subcore_axis_name="s")

    @functools.partial(
        pl.kernel,
        mesh=mesh,
        out_type=jax.ShapeDtypeStruct((nb, ns_pad, 2 * D), jnp.float32),
        scratch_types=[
            pltpu.VMEM((b_per_w * ns_pad,), jnp.int32),
            pltpu.VMEM((CH, 2 * D), jnp.float32),
            pltpu.SemaphoreType.DMA,
        ],
        compiler_params=pltpu.CompilerParams(needs_layout_passes=False),
    )
    def gather_kernel(t128_hbm, idx_hbm, out_hbm, idx_v, rows_v, sem):
        wid = lax.axis_index("s") * NC + lax.axis_index("c")
        base = wid * b_per_w * ns_pad
        pltpu.sync_copy(idx_hbm.at[pl.ds(base, b_per_w * ns_pad)], idx_v)

        def chunk(i, _):
            pltpu.async_copy(
                t128_hbm.at[idx_v.at[pl.ds(i * CH, CH)]], rows_v, sem
            ).wait()
            b0 = wid * b_per_w + i * GB
            for k in range(GB):
                pltpu.sync_copy(
                    rows_v.at[pl.ds(k * ns_pad, ns_pad)], out_hbm.at[b0 + k]
                )
            return _
        lax.fori_loop(0, n_ch, chunk, None)

    return gather_kernel


def kernel(table, inputs):
    nb, ns = inputs.shape
    D = table.shape[1]
    ns_pad = (ns + 7) // 8 * 8
    idxp = jnp.pad(inputs.astype(jnp.int32), ((0, 0), (0, ns_pad - ns)))
    idx = idxp.reshape(-1)
    V = table.shape[0]
    tT = table.T
    ntail = V % 512
    tailT = lax.slice(tT, (0, V - ntail), (D, V))
    t128 = _make_transpose(V)(tT, tailT)
    out3 = _make_gather(nb, ns_pad)(t128, idx)
    return out3[:, :ns, :D]


# speedup vs baseline: 1.7205x; 1.4917x over previous
import functools

import jax
import jax.numpy as jnp
from jax import lax
from jax.experimental import pallas as pl
from jax.experimental.pallas import tpu as pltpu
from jax.experimental.pallas import tpu_sc as plsc

EMBED_DIM = 64


@functools.lru_cache(maxsize=None)
def _make_gather(nb: int, ns_pad: int):
    D = EMBED_DIM
    info = plsc.get_sparse_core_info()
    NC, NS, L = info.num_cores, info.num_subcores, info.num_lanes
    NW = NC * NS
    assert nb % NW == 0
    b_per_w = nb // NW
    GB = 8
    CH = GB * ns_pad
    n_ch = b_per_w // GB
    B2 = nb * ns_pad

    mesh = plsc.VectorSubcoreMesh(core_axis_name="c", subcore_axis_name="s")

    @functools.partial(
        pl.kernel,
        mesh=mesh,
        out_type=jax.ShapeDtypeStruct((B2, 2 * D), jnp.float32),
        scratch_types=[
            pltpu.VMEM((b_per_w * ns_pad,), jnp.int32),
            pltpu.VMEM((CH, 2 * D), jnp.float32),
            pltpu.SemaphoreType.DMA,
        ],
        compiler_params=pltpu.CompilerParams(needs_layout_passes=False),
    )
    def gather_kernel(t128_hbm, idx_hbm, out_hbm, idx_v, rows_v, sem):
        wid = lax.axis_index("s") * NC + lax.axis_index("c")
        base = wid * b_per_w * ns_pad
        pltpu.sync_copy(idx_hbm.at[pl.ds(base, b_per_w * ns_pad)], idx_v)

        def chunk(i, _):
            pltpu.async_copy(
                t128_hbm.at[idx_v.at[pl.ds(i * CH, CH)]], rows_v, sem
            ).wait()
            pltpu.sync_copy(rows_v, out_hbm.at[pl.ds(base + i * CH, CH)])
            return _
        lax.fori_loop(0, n_ch, chunk, None)

    return gather_kernel


def kernel(table, inputs):
    nb, ns = inputs.shape
    D = table.shape[1]
    ns_pad = (ns + 7) // 8 * 8
    idx = jnp.pad(inputs.astype(jnp.int32), ((0, 0), (0, ns_pad - ns))).reshape(-1)
    table128 = jnp.pad(table, ((0, 0), (0, D)))
    out2 = _make_gather(nb, ns_pad)(table128, idx)
    return out2.reshape(nb, ns_pad, 2 * D)[:, :ns, :D]
